# baseline (speedup 1.0000x reference)
import functools

import jax
import jax.numpy as jnp
from jax import lax
from jax.experimental import pallas as pl
from jax.experimental.pallas import tpu as pltpu
from jax.experimental.pallas import tpu_sc as plsc

NUM_EXPERT = 16
TOP_K = 2
D_MODEL = 1024
D_FF = 1024
N_TOKENS = 8192

T = N_TOKENS
S = T * TOP_K
BM = 256
NB = S // BM + NUM_EXPERT
PMAX = NB * BM
SB = 2048
NSB = S // SB
NW = 32
SPT = S // NW
CB = 64
CH = SPT // CB

E_PAD = 128
NEG_BIG = -1e30


def _gating_body(x_ref, wg_ref, bg_ref, e1_ref, e2_ref, sc_ref):
    x = x_ref[...]
    logits = jnp.dot(x, wg_ref[...], preferred_element_type=jnp.float32)
    logits = logits + bg_ref[...]
    bt = logits.shape[0]
    lane = lax.broadcasted_iota(jnp.int32, (bt, E_PAD), 1)
    v1 = jnp.max(logits, axis=1, keepdims=True)
    i1 = jnp.min(jnp.where(logits >= v1, lane, E_PAD), axis=1, keepdims=True)
    oh1 = lane == i1
    logits2 = jnp.where(oh1, NEG_BIG, logits)
    v2 = jnp.max(logits2, axis=1, keepdims=True)
    i2 = jnp.min(jnp.where(logits2 >= v2, lane, E_PAD), axis=1, keepdims=True)
    e2v = jnp.exp(v2 - v1)
    s1 = 1.0 / (1.0 + e2v)
    s2 = 1.0 - s1
    zeros8 = jnp.zeros((bt, 8), jnp.int32)
    e1_ref[...] = i1 + zeros8
    e2_ref[...] = i2 + zeros8
    lane8 = lax.broadcasted_iota(jnp.int32, (bt, 8), 1)
    sc_ref[...] = jnp.where(lane8 == 0, s1, jnp.where(lane8 == 1, s2, 0.0))


def _gating(x, wg_pad, bg_pad, block_t=2048):
    return pl.pallas_call(
        _gating_body,
        grid=(T // block_t,),
        in_specs=[
            pl.BlockSpec((block_t, D_MODEL), lambda i: (i, 0)),
            pl.BlockSpec((D_MODEL, E_PAD), lambda i: (0, 0)),
            pl.BlockSpec((1, E_PAD), lambda i: (0, 0)),
        ],
        out_specs=[
            pl.BlockSpec((block_t, 8), lambda i: (i, 0)),
            pl.BlockSpec((block_t, 8), lambda i: (i, 0)),
            pl.BlockSpec((block_t, 8), lambda i: (i, 0)),
        ],
        out_shape=[
            jax.ShapeDtypeStruct((T, 8), jnp.int32),
            jax.ShapeDtypeStruct((T, 8), jnp.int32),
            jax.ShapeDtypeStruct((T, 8), jnp.float32),
        ],
    )(x, wg_pad, bg_pad)


def _rank_body(eids_ref, rank_ref, cnt_out_ref, cnt_ref):
    i = pl.program_id(0)

    @pl.when(i == 0)
    def _():
        cnt_ref[...] = jnp.zeros_like(cnt_ref)

    e_col = jnp.min(eids_ref[...], axis=1, keepdims=True)
    lane = lax.broadcasted_iota(jnp.int32, (SB, E_PAD), 1)
    oh = (lane == e_col).astype(jnp.float32)
    ohb = oh.astype(jnp.bfloat16)
    row = lax.broadcasted_iota(jnp.int32, (SB, SB), 0)
    col = lax.broadcasted_iota(jnp.int32, (SB, SB), 1)
    trilb = (row > col).astype(jnp.bfloat16)
    ranks_mat = jnp.dot(trilb, ohb, preferred_element_type=jnp.float32)
    rank = jnp.sum((ranks_mat + cnt_ref[...]) * oh, axis=1, keepdims=True)
    rank_ref[...] = rank + jnp.zeros((SB, 8), jnp.float32)
    cnt_ref[...] += jnp.sum(oh, axis=0, keepdims=True)
    cnt_out_ref[...] = cnt_ref[...]


def _rank(eids8):
    return pl.pallas_call(
        _rank_body,
        grid=(NSB,),
        in_specs=[pl.BlockSpec((SB, 8), lambda i: (i, 0))],
        out_specs=[
            pl.BlockSpec((SB, 8), lambda i: (i, 0)),
            pl.BlockSpec((1, E_PAD), lambda i: (0, 0)),
        ],
        out_shape=[
            jax.ShapeDtypeStruct((S, 8), jnp.float32),
            jax.ShapeDtypeStruct((1, E_PAD), jnp.float32),
        ],
        scratch_shapes=[pltpu.VMEM((1, E_PAD), jnp.float32)],
        compiler_params=pltpu.CompilerParams(
            dimension_semantics=("arbitrary",),
        ),
    )(eids8)


def _dest_body(eids_ref, rank_ref, cnt_ref, dest_ref, be_ref):
    cnt = cnt_ref[...]
    ac = jnp.floor((cnt + (BM - 1)) / BM) * BM
    sub = lax.broadcasted_iota(jnp.int32, (E_PAD, E_PAD), 0)
    lane2 = lax.broadcasted_iota(jnp.int32, (E_PAD, E_PAD), 1)
    mask_lt = (sub < lane2).astype(jnp.float32)
    off = jnp.dot(ac, mask_lt, preferred_element_type=jnp.float32)
    inccum = off + ac

    e_col = jnp.min(eids_ref[...], axis=1, keepdims=True)
    lane = lax.broadcasted_iota(jnp.int32, (SB, E_PAD), 1)
    oh = (lane == e_col).astype(jnp.float32)
    off_g = jnp.sum(off * oh, axis=1, keepdims=True)
    rank = jnp.min(rank_ref[...], axis=1, keepdims=True)
    dest = (rank + off_g).astype(jnp.int32)
    dest_ref[...] = dest + jnp.zeros((SB, 8), jnp.int32)

    eye = (sub == lane2).astype(jnp.float32)
    inc_col = jnp.sum(jnp.broadcast_to(inccum, (E_PAD, E_PAD)) * eye,
                      axis=1, keepdims=True)
    iblk = (lane2 * BM).astype(jnp.float32)
    m2 = jnp.where((inc_col <= iblk) & (sub < NUM_EXPERT), 1.0, 0.0)
    be = jnp.sum(m2, axis=0, keepdims=True)
    be_ref[...] = jnp.minimum(be, NUM_EXPERT - 1).astype(jnp.int32)


def _dest(eids8, rank8, counts):
    return pl.pallas_call(
        _dest_body,
        grid=(NSB,),
        in_specs=[
            pl.BlockSpec((SB, 8), lambda i: (i, 0)),
            pl.BlockSpec((SB, 8), lambda i: (i, 0)),
            pl.BlockSpec((1, E_PAD), lambda i: (0, 0)),
        ],
        out_specs=[
            pl.BlockSpec((SB, 8), lambda i: (i, 0)),
            pl.BlockSpec((1, E_PAD), lambda i: (0, 0)),
        ],
        out_shape=[
            jax.ShapeDtypeStruct((S, 8), jnp.int32),
            jax.ShapeDtypeStruct((1, E_PAD), jnp.int32),
        ],
    )(eids8, rank8, counts)


def _make_dispatch():
    mesh = plsc.VectorSubcoreMesh(core_axis_name="c", subcore_axis_name="s")

    @functools.partial(
        pl.kernel,
        mesh=mesh,
        out_type=jax.ShapeDtypeStruct((PMAX, D_MODEL), jnp.float32),
        scratch_types=[
            pltpu.VMEM((CH, CB), jnp.int32),
            pltpu.VMEM((CB, D_MODEL), jnp.float32),
            pltpu.SemaphoreType.DMA,
        ],
    )
    def disp(xb_hbm, dest_hbm, xs_hbm, idx_v, rows_v, sem):
        wid = lax.axis_index("s") * 2 + lax.axis_index("c")
        pltpu.sync_copy(dest_hbm.at[wid], idx_v)
        tok0 = (wid % (NW // TOP_K)) * SPT
        for c in range(CH):
            pltpu.sync_copy(xb_hbm.at[pl.ds(tok0 + c * CB, CB)], rows_v)
            pltpu.async_copy(rows_v, xs_hbm.at[idx_v.at[c]], sem).wait()

    return disp


def _make_combine():
    mesh = plsc.VectorSubcoreMesh(core_axis_name="c", subcore_axis_name="s")

    @functools.partial(
        pl.kernel,
        mesh=mesh,
        out_type=jax.ShapeDtypeStruct((S, D_MODEL), jnp.float32),
        scratch_types=[
            pltpu.VMEM((CH, CB), jnp.int32),
            pltpu.VMEM((CB, D_MODEL), jnp.float32),
            pltpu.SemaphoreType.DMA,
        ],
    )
    def comb(ys_hbm, dest_hbm, yus_hbm, idx_v, rows_v, sem):
        wid = lax.axis_index("s") * 2 + lax.axis_index("c")
        pltpu.sync_copy(dest_hbm.at[wid], idx_v)
        for c in range(CH):
            pltpu.async_copy(ys_hbm.at[idx_v.at[c]], rows_v, sem).wait()
            pltpu.sync_copy(rows_v, yus_hbm.at[pl.ds(wid * SPT + c * CB, CB)])

    return comb


def _ffn_body(be_ref, xs_ref, w1_ref, b1_ref, w2_ref, b2_ref, ys_ref):
    x = xs_ref[...].astype(jnp.bfloat16)
    h = jnp.dot(x, w1_ref[0], preferred_element_type=jnp.float32)
    h = jnp.maximum(h + b1_ref[0], 0.0).astype(jnp.bfloat16)
    y = jnp.dot(h, w2_ref[0], preferred_element_type=jnp.float32)
    ys_ref[...] = y + b2_ref[0]


def _ffn(be, xs, w1b, b1, w2b, b2):
    grid_spec = pltpu.PrefetchScalarGridSpec(
        num_scalar_prefetch=1,
        grid=(NB,),
        in_specs=[
            pl.BlockSpec((BM, D_MODEL), lambda i, be_r: (i, 0)),
            pl.BlockSpec((1, D_MODEL, D_FF), lambda i, be_r: (be_r[i], 0, 0)),
            pl.BlockSpec((1, 1, D_FF), lambda i, be_r: (be_r[i], 0, 0)),
            pl.BlockSpec((1, D_FF, D_MODEL), lambda i, be_r: (be_r[i], 0, 0)),
            pl.BlockSpec((1, 1, D_MODEL), lambda i, be_r: (be_r[i], 0, 0)),
        ],
        out_specs=pl.BlockSpec((BM, D_MODEL), lambda i, be_r: (i, 0)),
    )
    return pl.pallas_call(
        _ffn_body,
        grid_spec=grid_spec,
        out_shape=jax.ShapeDtypeStruct((PMAX, D_MODEL), jnp.float32),
        compiler_params=pltpu.CompilerParams(
            dimension_semantics=("arbitrary",),
        ),
    )(be, xs, w1b, b1, w2b, b2)


def _final_body(y1_ref, y2_ref, sc_ref, out_ref):
    bt = out_ref.shape[0]
    lane8 = lax.broadcasted_iota(jnp.int32, (bt, 8), 1)
    sc = sc_ref[...]
    s1 = jnp.sum(jnp.where(lane8 == 0, sc, 0.0), axis=1, keepdims=True)
    s2 = jnp.sum(jnp.where(lane8 == 1, sc, 0.0), axis=1, keepdims=True)
    out_ref[...] = s1 * y1_ref[...] + s2 * y2_ref[...]


def _final(yus, sc12, block_t=2048):
    nblk = T // block_t
    return pl.pallas_call(
        _final_body,
        grid=(nblk,),
        in_specs=[
            pl.BlockSpec((block_t, D_MODEL), lambda i: (i, 0)),
            pl.BlockSpec((block_t, D_MODEL), lambda i: (i + nblk, 0)),
            pl.BlockSpec((block_t, 8), lambda i: (i, 0)),
        ],
        out_specs=pl.BlockSpec((block_t, D_MODEL), lambda i: (i, 0)),
        out_shape=jax.ShapeDtypeStruct((T, D_MODEL), jnp.float32),
    )(yus, yus, sc12)


def kernel(x, Wg, bg, W1, b1, W2, b2):
    wg_pad = jnp.zeros((D_MODEL, E_PAD), jnp.float32).at[:, :NUM_EXPERT].set(Wg)
    bg_pad = jnp.full((1, E_PAD), NEG_BIG, jnp.float32).at[0, :NUM_EXPERT].set(bg)
    e1, e2, sc12 = _gating(x, wg_pad, bg_pad)
    eids8 = jnp.concatenate([e1, e2], axis=0)
    rank8, counts = _rank(eids8)
    dest8, blkexp = _dest(eids8, rank8, counts)
    dest = dest8[:, 0].reshape(NW, CH, CB)
    be = blkexp[0, :NB]

    xs = _make_dispatch()(x, dest)
    ys = _ffn(be, xs, W1.astype(jnp.bfloat16), b1[:, None, :],
              W2.astype(jnp.bfloat16), b2[:, None, :])
    yus = _make_combine()(ys, dest)
    return _final(yus, sc12)


# speedup vs baseline: 1.1174x; 1.1174x over previous
import functools

import jax
import jax.numpy as jnp
from jax import lax
from jax.experimental import pallas as pl
from jax.experimental.pallas import tpu as pltpu
from jax.experimental.pallas import tpu_sc as plsc

NUM_EXPERT = 16
TOP_K = 2
D_MODEL = 1024
D_FF = 1024
N_TOKENS = 8192

T = N_TOKENS
S = T * TOP_K
BM = 256
NB = S // BM + NUM_EXPERT
PMAX = NB * BM
SB = 2048
NSB = S // SB
NW = 32
SPT = S // NW
CB = 32
TOK_PT = T // NW
CHD = TOK_PT // CB
CHC = SPT // CB

E_PAD = 128
NEG_BIG = -1e30


def _gating_body(x_ref, wg_ref, bg_ref, e1_ref, e2_ref, sc_ref):
    x = x_ref[...]
    logits = jnp.dot(x, wg_ref[...], preferred_element_type=jnp.float32)
    logits = logits + bg_ref[...]
    bt = logits.shape[0]
    lane = lax.broadcasted_iota(jnp.int32, (bt, E_PAD), 1)
    v1 = jnp.max(logits, axis=1, keepdims=True)
    i1 = jnp.min(jnp.where(logits >= v1, lane, E_PAD), axis=1, keepdims=True)
    oh1 = lane == i1
    logits2 = jnp.where(oh1, NEG_BIG, logits)
    v2 = jnp.max(logits2, axis=1, keepdims=True)
    i2 = jnp.min(jnp.where(logits2 >= v2, lane, E_PAD), axis=1, keepdims=True)
    e2v = jnp.exp(v2 - v1)
    s1 = 1.0 / (1.0 + e2v)
    s2 = 1.0 - s1
    zeros8 = jnp.zeros((bt, 8), jnp.int32)
    e1_ref[...] = i1 + zeros8
    e2_ref[...] = i2 + zeros8
    lane8 = lax.broadcasted_iota(jnp.int32, (bt, 8), 1)
    sc_ref[...] = jnp.where(lane8 == 0, s1, jnp.where(lane8 == 1, s2, 0.0))


def _gating(x, wg_pad, bg_pad, block_t=2048):
    return pl.pallas_call(
        _gating_body,
        grid=(T // block_t,),
        in_specs=[
            pl.BlockSpec((block_t, D_MODEL), lambda i: (i, 0)),
            pl.BlockSpec((D_MODEL, E_PAD), lambda i: (0, 0)),
            pl.BlockSpec((1, E_PAD), lambda i: (0, 0)),
        ],
        out_specs=[
            pl.BlockSpec((block_t, 8), lambda i: (i, 0)),
            pl.BlockSpec((block_t, 8), lambda i: (i, 0)),
            pl.BlockSpec((block_t, 8), lambda i: (i, 0)),
        ],
        out_shape=[
            jax.ShapeDtypeStruct((T, 8), jnp.int32),
            jax.ShapeDtypeStruct((T, 8), jnp.int32),
            jax.ShapeDtypeStruct((T, 8), jnp.float32),
        ],
    )(x, wg_pad, bg_pad)


def _rank_body(eids_ref, rank_ref, cnt_out_ref, cnt_ref):
    i = pl.program_id(0)

    @pl.when(i == 0)
    def _():
        cnt_ref[...] = jnp.zeros_like(cnt_ref)

    e_col = jnp.min(eids_ref[...], axis=1, keepdims=True)
    lane = lax.broadcasted_iota(jnp.int32, (SB, E_PAD), 1)
    oh = (lane == e_col).astype(jnp.float32)
    ohb = oh.astype(jnp.bfloat16)
    row = lax.broadcasted_iota(jnp.int32, (SB, SB), 0)
    col = lax.broadcasted_iota(jnp.int32, (SB, SB), 1)
    trilb = (row > col).astype(jnp.bfloat16)
    ranks_mat = jnp.dot(trilb, ohb, preferred_element_type=jnp.float32)
    rank = jnp.sum((ranks_mat + cnt_ref[...]) * oh, axis=1, keepdims=True)
    rank_ref[...] = rank + jnp.zeros((SB, 8), jnp.float32)
    cnt_ref[...] += jnp.sum(oh, axis=0, keepdims=True)
    cnt_out_ref[...] = cnt_ref[...]


def _rank(eids8):
    return pl.pallas_call(
        _rank_body,
        grid=(NSB,),
        in_specs=[pl.BlockSpec((SB, 8), lambda i: (i, 0))],
        out_specs=[
            pl.BlockSpec((SB, 8), lambda i: (i, 0)),
            pl.BlockSpec((1, E_PAD), lambda i: (0, 0)),
        ],
        out_shape=[
            jax.ShapeDtypeStruct((S, 8), jnp.float32),
            jax.ShapeDtypeStruct((1, E_PAD), jnp.float32),
        ],
        scratch_shapes=[pltpu.VMEM((1, E_PAD), jnp.float32)],
        compiler_params=pltpu.CompilerParams(
            dimension_semantics=("arbitrary",),
        ),
    )(eids8)


def _dest_body(eids_ref, rank_ref, cnt_ref, dest_ref, be_ref):
    cnt = cnt_ref[...]
    ac = jnp.floor((cnt + (BM - 1)) / BM) * BM
    sub = lax.broadcasted_iota(jnp.int32, (E_PAD, E_PAD), 0)
    lane2 = lax.broadcasted_iota(jnp.int32, (E_PAD, E_PAD), 1)
    mask_lt = (sub < lane2).astype(jnp.float32)
    off = jnp.dot(ac, mask_lt, preferred_element_type=jnp.float32)
    inccum = off + ac

    e_col = jnp.min(eids_ref[...], axis=1, keepdims=True)
    lane = lax.broadcasted_iota(jnp.int32, (SB, E_PAD), 1)
    oh = (lane == e_col).astype(jnp.float32)
    off_g = jnp.sum(off * oh, axis=1, keepdims=True)
    rank = jnp.min(rank_ref[...], axis=1, keepdims=True)
    dest = (rank + off_g).astype(jnp.int32)
    dest_ref[...] = dest + jnp.zeros((SB, 8), jnp.int32)

    eye = (sub == lane2).astype(jnp.float32)
    inc_col = jnp.sum(jnp.broadcast_to(inccum, (E_PAD, E_PAD)) * eye,
                      axis=1, keepdims=True)
    iblk = (lane2 * BM).astype(jnp.float32)
    m2 = jnp.where((inc_col <= iblk) & (sub < NUM_EXPERT), 1.0, 0.0)
    be = jnp.sum(m2, axis=0, keepdims=True)
    be_ref[...] = jnp.minimum(be, NUM_EXPERT - 1).astype(jnp.int32)


def _dest(eids8, rank8, counts):
    return pl.pallas_call(
        _dest_body,
        grid=(NSB,),
        in_specs=[
            pl.BlockSpec((SB, 8), lambda i: (i, 0)),
            pl.BlockSpec((SB, 8), lambda i: (i, 0)),
            pl.BlockSpec((1, E_PAD), lambda i: (0, 0)),
        ],
        out_specs=[
            pl.BlockSpec((SB, 8), lambda i: (i, 0)),
            pl.BlockSpec((1, E_PAD), lambda i: (0, 0)),
        ],
        out_shape=[
            jax.ShapeDtypeStruct((S, 8), jnp.int32),
            jax.ShapeDtypeStruct((1, E_PAD), jnp.int32),
        ],
    )(eids8, rank8, counts)


def _make_dispatch():
    mesh = plsc.VectorSubcoreMesh(core_axis_name="c", subcore_axis_name="s")

    @functools.partial(
        pl.kernel,
        mesh=mesh,
        out_type=jax.ShapeDtypeStruct((PMAX, D_MODEL), jnp.float32),
        scratch_types=[
            pltpu.VMEM((2 * CHD, CB), jnp.int32),
            pltpu.VMEM((CB, D_MODEL), jnp.float32),
            pltpu.VMEM((CB, D_MODEL), jnp.float32),
            pltpu.SemaphoreType.DMA,
            pltpu.SemaphoreType.DMA,
        ],
    )
    def disp(xb_hbm, dest_hbm, xs_hbm, idx_v, rows_a, rows_b, sem_r, sem_w):
        wid = lax.axis_index("s") * 2 + lax.axis_index("c")
        pltpu.sync_copy(dest_hbm.at[wid], idx_v)
        tok0 = wid * TOK_PT
        bufs = [rows_a, rows_b]
        rd = pltpu.async_copy(xb_hbm.at[pl.ds(tok0, CB)], rows_a, sem_r)
        prev_w = None
        for c in range(CHD):
            cur = bufs[c % 2]
            rd.wait()
            if prev_w is not None:
                prev_w[0].wait()
                prev_w[1].wait()
            if c + 1 < CHD:
                rd = pltpu.async_copy(
                    xb_hbm.at[pl.ds(tok0 + (c + 1) * CB, CB)],
                    bufs[(c + 1) % 2], sem_r)
            w1 = pltpu.async_copy(cur, xs_hbm.at[idx_v.at[c]], sem_w)
            w2 = pltpu.async_copy(cur, xs_hbm.at[idx_v.at[CHD + c]], sem_w)
            prev_w = (w1, w2)
        prev_w[0].wait()
        prev_w[1].wait()

    return disp


def _make_combine():
    mesh = plsc.VectorSubcoreMesh(core_axis_name="c", subcore_axis_name="s")

    @functools.partial(
        pl.kernel,
        mesh=mesh,
        out_type=jax.ShapeDtypeStruct((S, D_MODEL), jnp.float32),
        scratch_types=[
            pltpu.VMEM((CHC, CB), jnp.int32),
            pltpu.VMEM((CB, D_MODEL), jnp.float32),
            pltpu.VMEM((CB, D_MODEL), jnp.float32),
            pltpu.SemaphoreType.DMA,
            pltpu.SemaphoreType.DMA,
        ],
    )
    def comb(ys_hbm, dest_hbm, yus_hbm, idx_v, rows_a, rows_b, sem_r, sem_w):
        wid = lax.axis_index("s") * 2 + lax.axis_index("c")
        pltpu.sync_copy(dest_hbm.at[wid], idx_v)
        slot0 = wid * SPT
        bufs = [rows_a, rows_b]
        rd = pltpu.async_copy(ys_hbm.at[idx_v.at[0]], rows_a, sem_r)
        prev_w = None
        for c in range(CHC):
            cur = bufs[c % 2]
            rd.wait()
            if prev_w is not None:
                prev_w.wait()
            if c + 1 < CHC:
                rd = pltpu.async_copy(ys_hbm.at[idx_v.at[c + 1]],
                                      bufs[(c + 1) % 2], sem_r)
            prev_w = pltpu.async_copy(
                cur, yus_hbm.at[pl.ds(slot0 + c * CB, CB)], sem_w)
        prev_w.wait()

    return comb


def _ffn_body(be_ref, xs_ref, w1_ref, b1_ref, w2_ref, b2_ref, ys_ref):
    x = xs_ref[...].astype(jnp.bfloat16)
    w1 = w1_ref[0].astype(jnp.bfloat16)
    h = jnp.dot(x, w1, preferred_element_type=jnp.float32)
    h = jnp.maximum(h + b1_ref[0], 0.0).astype(jnp.bfloat16)
    w2 = w2_ref[0].astype(jnp.bfloat16)
    y = jnp.dot(h, w2, preferred_element_type=jnp.float32)
    ys_ref[...] = y + b2_ref[0]


def _ffn(be, xs, w1b, b1, w2b, b2):
    grid_spec = pltpu.PrefetchScalarGridSpec(
        num_scalar_prefetch=1,
        grid=(NB,),
        in_specs=[
            pl.BlockSpec((BM, D_MODEL), lambda i, be_r: (i, 0)),
            pl.BlockSpec((1, D_MODEL, D_FF), lambda i, be_r: (be_r[i], 0, 0)),
            pl.BlockSpec((1, 1, D_FF), lambda i, be_r: (be_r[i], 0, 0)),
            pl.BlockSpec((1, D_FF, D_MODEL), lambda i, be_r: (be_r[i], 0, 0)),
            pl.BlockSpec((1, 1, D_MODEL), lambda i, be_r: (be_r[i], 0, 0)),
        ],
        out_specs=pl.BlockSpec((BM, D_MODEL), lambda i, be_r: (i, 0)),
    )
    return pl.pallas_call(
        _ffn_body,
        grid_spec=grid_spec,
        out_shape=jax.ShapeDtypeStruct((PMAX, D_MODEL), jnp.float32),
        compiler_params=pltpu.CompilerParams(
            dimension_semantics=("arbitrary",),
        ),
    )(be, xs, w1b, b1, w2b, b2)


def _final_body(y1_ref, y2_ref, sc_ref, out_ref):
    bt = out_ref.shape[0]
    lane8 = lax.broadcasted_iota(jnp.int32, (bt, 8), 1)
    sc = sc_ref[...]
    s1 = jnp.sum(jnp.where(lane8 == 0, sc, 0.0), axis=1, keepdims=True)
    s2 = jnp.sum(jnp.where(lane8 == 1, sc, 0.0), axis=1, keepdims=True)
    out_ref[...] = s1 * y1_ref[...] + s2 * y2_ref[...]


def _final(yus, sc12, block_t=2048):
    nblk = T // block_t
    return pl.pallas_call(
        _final_body,
        grid=(nblk,),
        in_specs=[
            pl.BlockSpec((block_t, D_MODEL), lambda i: (i, 0)),
            pl.BlockSpec((block_t, D_MODEL), lambda i: (i + nblk, 0)),
            pl.BlockSpec((block_t, 8), lambda i: (i, 0)),
        ],
        out_specs=pl.BlockSpec((block_t, D_MODEL), lambda i: (i, 0)),
        out_shape=jax.ShapeDtypeStruct((T, D_MODEL), jnp.float32),
    )(yus, yus, sc12)


def kernel(x, Wg, bg, W1, b1, W2, b2):
    wg_pad = jnp.zeros((D_MODEL, E_PAD), jnp.float32).at[:, :NUM_EXPERT].set(Wg)
    bg_pad = jnp.full((1, E_PAD), NEG_BIG, jnp.float32).at[0, :NUM_EXPERT].set(bg)
    e1, e2, sc12 = _gating(x, wg_pad, bg_pad)
    eids8 = jnp.concatenate([e1, e2], axis=0)
    rank8, counts = _rank(eids8)
    dest8, blkexp = _dest(eids8, rank8, counts)
    dcol = dest8[:, 0]
    dlo = dcol[:T].reshape(NW, CHD, CB)
    dhi = dcol[T:].reshape(NW, CHD, CB)
    dpair = jnp.concatenate([dlo, dhi], axis=1)
    dcomb = dcol.reshape(NW, CHC, CB)
    be = blkexp[0, :NB]

    xs = _make_dispatch()(x, dpair)
    ys = _ffn(be, xs, W1, b1[:, None, :], W2, b2[:, None, :])
    yus = _make_combine()(ys, dcomb)
    return _final(yus, sc12)


# speedup vs baseline: 1.2065x; 1.0797x over previous
import functools

import jax
import jax.numpy as jnp
from jax import lax
from jax.experimental import pallas as pl
from jax.experimental.pallas import tpu as pltpu
from jax.experimental.pallas import tpu_sc as plsc

NUM_EXPERT = 16
TOP_K = 2
D_MODEL = 1024
D_FF = 1024
N_TOKENS = 8192

T = N_TOKENS
S = T * TOP_K
BM = 256
NB = S // BM + NUM_EXPERT
PMAX = NB * BM
SB = 2048
NSB = S // SB
NW = 32
SPT = S // NW
CB = 32
TOK_PT = T // NW
CHD = TOK_PT // CB
CBC = 16
CHT = TOK_PT // CBC
WSW = 16

E_PAD = 128
NEG_BIG = -1e30


def _gating_body(x_ref, wg_ref, bg_ref, e1_ref, e2_ref, sc_ref):
    x = x_ref[...]
    logits = jnp.dot(x, wg_ref[...], preferred_element_type=jnp.float32)
    logits = logits + bg_ref[...]
    bt = logits.shape[0]
    lane = lax.broadcasted_iota(jnp.int32, (bt, E_PAD), 1)
    v1 = jnp.max(logits, axis=1, keepdims=True)
    i1 = jnp.min(jnp.where(logits >= v1, lane, E_PAD), axis=1, keepdims=True)
    oh1 = lane == i1
    logits2 = jnp.where(oh1, NEG_BIG, logits)
    v2 = jnp.max(logits2, axis=1, keepdims=True)
    i2 = jnp.min(jnp.where(logits2 >= v2, lane, E_PAD), axis=1, keepdims=True)
    e2v = jnp.exp(v2 - v1)
    s1 = 1.0 / (1.0 + e2v)
    s2 = 1.0 - s1
    zeros8 = jnp.zeros((bt, 8), jnp.int32)
    e1_ref[...] = i1 + zeros8
    e2_ref[...] = i2 + zeros8
    lane8 = lax.broadcasted_iota(jnp.int32, (bt, 8), 1)
    sc_ref[...] = jnp.where(lane8 == 0, s1, jnp.where(lane8 == 1, s2, 0.0))


def _gating(x, wg_pad, bg_pad, block_t=2048):
    return pl.pallas_call(
        _gating_body,
        grid=(T // block_t,),
        in_specs=[
            pl.BlockSpec((block_t, D_MODEL), lambda i: (i, 0)),
            pl.BlockSpec((D_MODEL, E_PAD), lambda i: (0, 0)),
            pl.BlockSpec((1, E_PAD), lambda i: (0, 0)),
        ],
        out_specs=[
            pl.BlockSpec((block_t, 8), lambda i: (i, 0)),
            pl.BlockSpec((block_t, 8), lambda i: (i, 0)),
            pl.BlockSpec((block_t, 8), lambda i: (i, 0)),
        ],
        out_shape=[
            jax.ShapeDtypeStruct((T, 8), jnp.int32),
            jax.ShapeDtypeStruct((T, 8), jnp.int32),
            jax.ShapeDtypeStruct((T, 8), jnp.float32),
        ],
    )(x, wg_pad, bg_pad)


def _rank_body(eids_ref, rank_ref, cnt_out_ref, cnt_ref):
    i = pl.program_id(0)

    @pl.when(i == 0)
    def _():
        cnt_ref[...] = jnp.zeros_like(cnt_ref)

    e_col = jnp.min(eids_ref[...], axis=1, keepdims=True)
    lane = lax.broadcasted_iota(jnp.int32, (SB, E_PAD), 1)
    oh = (lane == e_col).astype(jnp.float32)
    ohb = oh.astype(jnp.bfloat16)
    row = lax.broadcasted_iota(jnp.int32, (SB, SB), 0)
    col = lax.broadcasted_iota(jnp.int32, (SB, SB), 1)
    trilb = (row > col).astype(jnp.bfloat16)
    ranks_mat = jnp.dot(trilb, ohb, preferred_element_type=jnp.float32)
    rank = jnp.sum((ranks_mat + cnt_ref[...]) * oh, axis=1, keepdims=True)
    rank_ref[...] = rank + jnp.zeros((SB, 8), jnp.float32)
    cnt_ref[...] += jnp.sum(oh, axis=0, keepdims=True)
    cnt_out_ref[...] = cnt_ref[...]


def _rank(eids8):
    return pl.pallas_call(
        _rank_body,
        grid=(NSB,),
        in_specs=[pl.BlockSpec((SB, 8), lambda i: (i, 0))],
        out_specs=[
            pl.BlockSpec((SB, 8), lambda i: (i, 0)),
            pl.BlockSpec((1, E_PAD), lambda i: (0, 0)),
        ],
        out_shape=[
            jax.ShapeDtypeStruct((S, 8), jnp.float32),
            jax.ShapeDtypeStruct((1, E_PAD), jnp.float32),
        ],
        scratch_shapes=[pltpu.VMEM((1, E_PAD), jnp.float32)],
        compiler_params=pltpu.CompilerParams(
            dimension_semantics=("arbitrary",),
        ),
    )(eids8)


def _dest_body(eids_ref, rank_ref, cnt_ref, dest_ref, be_ref):
    cnt = cnt_ref[...]
    ac = jnp.floor((cnt + (BM - 1)) / BM) * BM
    sub = lax.broadcasted_iota(jnp.int32, (E_PAD, E_PAD), 0)
    lane2 = lax.broadcasted_iota(jnp.int32, (E_PAD, E_PAD), 1)
    mask_lt = (sub < lane2).astype(jnp.float32)
    off = jnp.dot(ac, mask_lt, preferred_element_type=jnp.float32)
    inccum = off + ac

    e_col = jnp.min(eids_ref[...], axis=1, keepdims=True)
    lane = lax.broadcasted_iota(jnp.int32, (SB, E_PAD), 1)
    oh = (lane == e_col).astype(jnp.float32)
    off_g = jnp.sum(off * oh, axis=1, keepdims=True)
    rank = jnp.min(rank_ref[...], axis=1, keepdims=True)
    dest = (rank + off_g).astype(jnp.int32)
    dest_ref[...] = dest + jnp.zeros((SB, 8), jnp.int32)

    eye = (sub == lane2).astype(jnp.float32)
    inc_col = jnp.sum(jnp.broadcast_to(inccum, (E_PAD, E_PAD)) * eye,
                      axis=1, keepdims=True)
    iblk = (lane2 * BM).astype(jnp.float32)
    m2 = jnp.where((inc_col <= iblk) & (sub < NUM_EXPERT), 1.0, 0.0)
    be = jnp.sum(m2, axis=0, keepdims=True)
    be_ref[...] = jnp.minimum(be, NUM_EXPERT - 1).astype(jnp.int32)


def _dest(eids8, rank8, counts):
    return pl.pallas_call(
        _dest_body,
        grid=(NSB,),
        in_specs=[
            pl.BlockSpec((SB, 8), lambda i: (i, 0)),
            pl.BlockSpec((SB, 8), lambda i: (i, 0)),
            pl.BlockSpec((1, E_PAD), lambda i: (0, 0)),
        ],
        out_specs=[
            pl.BlockSpec((SB, 8), lambda i: (i, 0)),
            pl.BlockSpec((1, E_PAD), lambda i: (0, 0)),
        ],
        out_shape=[
            jax.ShapeDtypeStruct((S, 8), jnp.int32),
            jax.ShapeDtypeStruct((1, E_PAD), jnp.int32),
        ],
    )(eids8, rank8, counts)


def _make_dispatch():
    mesh = plsc.VectorSubcoreMesh(core_axis_name="c", subcore_axis_name="s")

    @functools.partial(
        pl.kernel,
        mesh=mesh,
        out_type=jax.ShapeDtypeStruct((PMAX, D_MODEL), jnp.float32),
        scratch_types=[
            pltpu.VMEM((2 * CHD, CB), jnp.int32),
            pltpu.VMEM((CB, D_MODEL), jnp.float32),
            pltpu.VMEM((CB, D_MODEL), jnp.float32),
            pltpu.SemaphoreType.DMA,
            pltpu.SemaphoreType.DMA,
        ],
    )
    def disp(xb_hbm, dest_hbm, xs_hbm, idx_v, rows_a, rows_b, sem_r, sem_w):
        wid = lax.axis_index("s") * 2 + lax.axis_index("c")
        pltpu.sync_copy(dest_hbm.at[wid], idx_v)
        tok0 = wid * TOK_PT
        bufs = [rows_a, rows_b]
        rd = pltpu.async_copy(xb_hbm.at[pl.ds(tok0, CB)], rows_a, sem_r)
        prev_w = None
        for c in range(CHD):
            cur = bufs[c % 2]
            rd.wait()
            if prev_w is not None:
                prev_w[0].wait()
                prev_w[1].wait()
            if c + 1 < CHD:
                rd = pltpu.async_copy(
                    xb_hbm.at[pl.ds(tok0 + (c + 1) * CB, CB)],
                    bufs[(c + 1) % 2], sem_r)
            w1 = pltpu.async_copy(cur, xs_hbm.at[idx_v.at[c]], sem_w)
            w2 = pltpu.async_copy(cur, xs_hbm.at[idx_v.at[CHD + c]], sem_w)
            prev_w = (w1, w2)
        prev_w[0].wait()
        prev_w[1].wait()

    return disp


def _make_combine():
    mesh = plsc.VectorSubcoreMesh(core_axis_name="c", subcore_axis_name="s")

    @functools.partial(
        pl.kernel,
        mesh=mesh,
        out_type=jax.ShapeDtypeStruct((T, D_MODEL), jnp.float32),
        scratch_types=[
            pltpu.VMEM((2 * CHT, CBC), jnp.int32),
            pltpu.VMEM((TOK_PT, 16), jnp.float32),
            pltpu.VMEM((CBC, D_MODEL), jnp.float32),
            pltpu.VMEM((CBC, D_MODEL), jnp.float32),
            pltpu.VMEM((CBC, D_MODEL), jnp.float32),
            pltpu.VMEM((CBC, D_MODEL), jnp.float32),
            pltpu.VMEM((CBC, D_MODEL), jnp.float32),
            pltpu.SemaphoreType.DMA,
            pltpu.SemaphoreType.DMA,
        ],
    )
    def comb(ys_hbm, dest_hbm, s1_hbm, out_hbm, idx_v, s1_v,
             r1a, r1b, r2a, r2b, oa, sem_r, sem_w):
        wid = lax.axis_index("s") * 2 + lax.axis_index("c")
        tok0 = wid * TOK_PT
        pltpu.sync_copy(dest_hbm.at[wid], idx_v)
        pltpu.sync_copy(s1_hbm.at[pl.ds(tok0, TOK_PT)], s1_v)
        bufs1 = [r1a, r1b]
        bufs2 = [r2a, r2b]
        g1 = pltpu.async_copy(ys_hbm.at[idx_v.at[0]], r1a, sem_r)
        g2 = pltpu.async_copy(ys_hbm.at[idx_v.at[CHT]], r2a, sem_r)
        wout = None
        for c in range(CHT):
            cur1 = bufs1[c % 2]
            cur2 = bufs2[c % 2]
            g1.wait()
            g2.wait()
            if c + 1 < CHT:
                g1 = pltpu.async_copy(ys_hbm.at[idx_v.at[c + 1]],
                                      bufs1[(c + 1) % 2], sem_r)
                g2 = pltpu.async_copy(ys_hbm.at[idx_v.at[CHT + c + 1]],
                                      bufs2[(c + 1) % 2], sem_r)
            if wout is not None:
                wout.wait()

            def body(r, _):
                s1v = s1_v[c * CBC + r, pl.ds(0, 16)]
                for v in range(D_MODEL // 16):
                    a = cur1[r, pl.ds(v * 16, 16)]
                    b = cur2[r, pl.ds(v * 16, 16)]
                    oa[r, pl.ds(v * 16, 16)] = b + s1v * (a - b)
                return 0

            lax.fori_loop(0, CBC, body, 0)
            wout = pltpu.async_copy(
                oa, out_hbm.at[pl.ds(tok0 + c * CBC, CBC)], sem_w)
        wout.wait()

    return comb


def _ffn_body(be_ref, xs_ref, w1_ref, b1_ref, w2_ref, b2_ref, ys_ref):
    x = xs_ref[...].astype(jnp.bfloat16)
    w1 = w1_ref[0].astype(jnp.bfloat16)
    h = jnp.dot(x, w1, preferred_element_type=jnp.float32)
    h = jnp.maximum(h + b1_ref[0], 0.0).astype(jnp.bfloat16)
    w2 = w2_ref[0].astype(jnp.bfloat16)
    y = jnp.dot(h, w2, preferred_element_type=jnp.float32)
    ys_ref[...] = y + b2_ref[0]


def _ffn(be, xs, w1b, b1, w2b, b2):
    grid_spec = pltpu.PrefetchScalarGridSpec(
        num_scalar_prefetch=1,
        grid=(NB,),
        in_specs=[
            pl.BlockSpec((BM, D_MODEL), lambda i, be_r: (i, 0)),
            pl.BlockSpec((1, D_MODEL, D_FF), lambda i, be_r: (be_r[i], 0, 0)),
            pl.BlockSpec((1, 1, D_FF), lambda i, be_r: (be_r[i], 0, 0)),
            pl.BlockSpec((1, D_FF, D_MODEL), lambda i, be_r: (be_r[i], 0, 0)),
            pl.BlockSpec((1, 1, D_MODEL), lambda i, be_r: (be_r[i], 0, 0)),
        ],
        out_specs=pl.BlockSpec((BM, D_MODEL), lambda i, be_r: (i, 0)),
    )
    return pl.pallas_call(
        _ffn_body,
        grid_spec=grid_spec,
        out_shape=jax.ShapeDtypeStruct((PMAX, D_MODEL), jnp.float32),
        compiler_params=pltpu.CompilerParams(
            dimension_semantics=("arbitrary",),
        ),
    )(be, xs, w1b, b1, w2b, b2)


def kernel(x, Wg, bg, W1, b1, W2, b2):
    wg_pad = jnp.zeros((D_MODEL, E_PAD), jnp.float32).at[:, :NUM_EXPERT].set(Wg)
    bg_pad = jnp.full((1, E_PAD), NEG_BIG, jnp.float32).at[0, :NUM_EXPERT].set(bg)
    e1, e2, sc12 = _gating(x, wg_pad, bg_pad)
    eids8 = jnp.concatenate([e1, e2], axis=0)
    rank8, counts = _rank(eids8)
    dest8, blkexp = _dest(eids8, rank8, counts)
    dcol = dest8[:, 0]
    dlo = dcol[:T]
    dhi = dcol[T:]
    dpair = jnp.concatenate([dlo.reshape(NW, CHD, CB),
                             dhi.reshape(NW, CHD, CB)], axis=1)
    dcomb = jnp.concatenate([dlo.reshape(NW, CHT, CBC),
                             dhi.reshape(NW, CHT, CBC)], axis=1)
    s1b = jnp.broadcast_to(sc12[:, 0:1], (T, 16))
    be = blkexp[0, :NB]

    xs = _make_dispatch()(x, dpair)
    ys = _ffn(be, xs, W1, b1[:, None, :], W2, b2[:, None, :])
    return _make_combine()(ys, dcomb, s1b)


# speedup vs baseline: 1.2078x; 1.0011x over previous
import functools

import jax
import jax.numpy as jnp
from jax import lax
from jax.experimental import pallas as pl
from jax.experimental.pallas import tpu as pltpu
from jax.experimental.pallas import tpu_sc as plsc

NUM_EXPERT = 16
TOP_K = 2
D_MODEL = 1024
D_FF = 1024
N_TOKENS = 8192

T = N_TOKENS
S = T * TOP_K
BM = 256
NB = S // BM + NUM_EXPERT
PMAX = NB * BM
SB = 2048
NSB = S // SB
NW = 32
SPT = S // NW
CB = 32
TOK_PT = T // NW
CHD = TOK_PT // CB
CBC = 16
CHT = TOK_PT // CBC

E_PAD = 128
NEG_BIG = -1e30


def _gating_body(x_ref, wg_ref, bg_ref, e1_ref, e2_ref, sc_ref):
    x = x_ref[...]
    logits = jnp.dot(x, wg_ref[...], preferred_element_type=jnp.float32)
    logits = logits + bg_ref[...]
    bt = logits.shape[0]
    lane = lax.broadcasted_iota(jnp.int32, (bt, E_PAD), 1)
    v1 = jnp.max(logits, axis=1, keepdims=True)
    i1 = jnp.min(jnp.where(logits >= v1, lane, E_PAD), axis=1, keepdims=True)
    oh1 = lane == i1
    logits2 = jnp.where(oh1, NEG_BIG, logits)
    v2 = jnp.max(logits2, axis=1, keepdims=True)
    i2 = jnp.min(jnp.where(logits2 >= v2, lane, E_PAD), axis=1, keepdims=True)
    e2v = jnp.exp(v2 - v1)
    s1 = 1.0 / (1.0 + e2v)
    s2 = 1.0 - s1
    zeros8 = jnp.zeros((bt, 8), jnp.int32)
    e1_ref[...] = i1 + zeros8
    e2_ref[...] = i2 + zeros8
    lane8 = lax.broadcasted_iota(jnp.int32, (bt, 8), 1)
    sc_ref[...] = jnp.where(lane8 == 0, s1, jnp.where(lane8 == 1, s2, 0.0))


def _gating(x, wg_pad, bg_pad, block_t=2048):
    return pl.pallas_call(
        _gating_body,
        grid=(T // block_t,),
        in_specs=[
            pl.BlockSpec((block_t, D_MODEL), lambda i: (i, 0)),
            pl.BlockSpec((D_MODEL, E_PAD), lambda i: (0, 0)),
            pl.BlockSpec((1, E_PAD), lambda i: (0, 0)),
        ],
        out_specs=[
            pl.BlockSpec((block_t, 8), lambda i: (i, 0)),
            pl.BlockSpec((block_t, 8), lambda i: (i, 0)),
            pl.BlockSpec((block_t, 8), lambda i: (i, 0)),
        ],
        out_shape=[
            jax.ShapeDtypeStruct((T, 8), jnp.int32),
            jax.ShapeDtypeStruct((T, 8), jnp.int32),
            jax.ShapeDtypeStruct((T, 8), jnp.float32),
        ],
    )(x, wg_pad, bg_pad)


def _rank_body(eids_ref, rank_ref, cnt_out_ref, cnt_ref):
    i = pl.program_id(0)

    @pl.when(i == 0)
    def _():
        cnt_ref[...] = jnp.zeros_like(cnt_ref)

    e_col = jnp.min(eids_ref[...], axis=1, keepdims=True)
    lane = lax.broadcasted_iota(jnp.int32, (SB, E_PAD), 1)
    oh = (lane == e_col).astype(jnp.float32)
    ohb = oh.astype(jnp.bfloat16)
    row = lax.broadcasted_iota(jnp.int32, (SB, SB), 0)
    col = lax.broadcasted_iota(jnp.int32, (SB, SB), 1)
    trilb = (row > col).astype(jnp.bfloat16)
    ranks_mat = jnp.dot(trilb, ohb, preferred_element_type=jnp.float32)
    rank = jnp.sum((ranks_mat + cnt_ref[...]) * oh, axis=1, keepdims=True)
    rank_ref[...] = rank + jnp.zeros((SB, 8), jnp.float32)
    cnt_ref[...] += jnp.sum(oh, axis=0, keepdims=True)
    cnt_out_ref[...] = cnt_ref[...]


def _rank(eids8):
    return pl.pallas_call(
        _rank_body,
        grid=(NSB,),
        in_specs=[pl.BlockSpec((SB, 8), lambda i: (i, 0))],
        out_specs=[
            pl.BlockSpec((SB, 8), lambda i: (i, 0)),
            pl.BlockSpec((1, E_PAD), lambda i: (0, 0)),
        ],
        out_shape=[
            jax.ShapeDtypeStruct((S, 8), jnp.float32),
            jax.ShapeDtypeStruct((1, E_PAD), jnp.float32),
        ],
        scratch_shapes=[pltpu.VMEM((1, E_PAD), jnp.float32)],
        compiler_params=pltpu.CompilerParams(
            dimension_semantics=("arbitrary",),
        ),
    )(eids8)


def _dest_body(eids_ref, rank_ref, cnt_ref, dest_ref, be_ref):
    cnt = cnt_ref[...]
    ac = jnp.floor((cnt + (BM - 1)) / BM) * BM
    sub = lax.broadcasted_iota(jnp.int32, (E_PAD, E_PAD), 0)
    lane2 = lax.broadcasted_iota(jnp.int32, (E_PAD, E_PAD), 1)
    mask_lt = (sub < lane2).astype(jnp.float32)
    off = jnp.dot(ac, mask_lt, preferred_element_type=jnp.float32)
    inccum = off + ac

    e_col = jnp.min(eids_ref[...], axis=1, keepdims=True)
    lane = lax.broadcasted_iota(jnp.int32, (SB, E_PAD), 1)
    oh = (lane == e_col).astype(jnp.float32)
    off_g = jnp.sum(off * oh, axis=1, keepdims=True)
    rank = jnp.min(rank_ref[...], axis=1, keepdims=True)
    dest = (rank + off_g).astype(jnp.int32)
    dest_ref[...] = dest + jnp.zeros((SB, 8), jnp.int32)

    eye = (sub == lane2).astype(jnp.float32)
    inc_col = jnp.sum(jnp.broadcast_to(inccum, (E_PAD, E_PAD)) * eye,
                      axis=1, keepdims=True)
    iblk = (lane2 * BM).astype(jnp.float32)
    m2 = jnp.where((inc_col <= iblk) & (sub < NUM_EXPERT), 1.0, 0.0)
    be = jnp.sum(m2, axis=0, keepdims=True)
    be_ref[...] = jnp.minimum(be, NUM_EXPERT - 1).astype(jnp.int32)


def _dest(eids8, rank8, counts):
    return pl.pallas_call(
        _dest_body,
        grid=(NSB,),
        in_specs=[
            pl.BlockSpec((SB, 8), lambda i: (i, 0)),
            pl.BlockSpec((SB, 8), lambda i: (i, 0)),
            pl.BlockSpec((1, E_PAD), lambda i: (0, 0)),
        ],
        out_specs=[
            pl.BlockSpec((SB, 8), lambda i: (i, 0)),
            pl.BlockSpec((1, E_PAD), lambda i: (0, 0)),
        ],
        out_shape=[
            jax.ShapeDtypeStruct((S, 8), jnp.int32),
            jax.ShapeDtypeStruct((1, E_PAD), jnp.int32),
        ],
    )(eids8, rank8, counts)


def _make_dispatch():
    mesh = plsc.VectorSubcoreMesh(core_axis_name="c", subcore_axis_name="s")

    @functools.partial(
        pl.kernel,
        mesh=mesh,
        out_type=jax.ShapeDtypeStruct((PMAX, D_MODEL), jnp.float32),
        scratch_types=[
            pltpu.VMEM((2 * CHD, CB), jnp.int32),
            pltpu.VMEM((CB, D_MODEL), jnp.float32),
            pltpu.VMEM((CB, D_MODEL), jnp.float32),
            pltpu.SemaphoreType.DMA,
            pltpu.SemaphoreType.DMA,
        ],
    )
    def disp(xb_hbm, dest_hbm, xs_hbm, idx_v, rows_a, rows_b, sem_r, sem_w):
        wid = lax.axis_index("s") * 2 + lax.axis_index("c")
        pltpu.sync_copy(dest_hbm.at[wid], idx_v)
        tok0 = wid * TOK_PT
        bufs = [rows_a, rows_b]
        rd = pltpu.async_copy(xb_hbm.at[pl.ds(tok0, CB)], rows_a, sem_r)
        prev_w = None
        for c in range(CHD):
            cur = bufs[c % 2]
            rd.wait()
            if prev_w is not None:
                prev_w[0].wait()
                prev_w[1].wait()
            if c + 1 < CHD:
                rd = pltpu.async_copy(
                    xb_hbm.at[pl.ds(tok0 + (c + 1) * CB, CB)],
                    bufs[(c + 1) % 2], sem_r)
            w1 = pltpu.async_copy(cur, xs_hbm.at[idx_v.at[c]], sem_w)
            w2 = pltpu.async_copy(cur, xs_hbm.at[idx_v.at[CHD + c]], sem_w)
            prev_w = (w1, w2)
        prev_w[0].wait()
        prev_w[1].wait()

    return disp


def _make_combine():
    mesh = plsc.VectorSubcoreMesh(core_axis_name="c", subcore_axis_name="s")

    @functools.partial(
        pl.kernel,
        mesh=mesh,
        out_type=jax.ShapeDtypeStruct((T, D_MODEL), jnp.float32),
        scratch_types=[
            pltpu.VMEM((2 * CHT, CBC), jnp.int32),
            pltpu.VMEM((TOK_PT, 16), jnp.float32),
            pltpu.VMEM((CBC, D_MODEL), jnp.float32),
            pltpu.VMEM((CBC, D_MODEL), jnp.float32),
            pltpu.VMEM((CBC, D_MODEL), jnp.float32),
            pltpu.VMEM((CBC, D_MODEL), jnp.float32),
            pltpu.VMEM((CBC, D_MODEL), jnp.float32),
            pltpu.SemaphoreType.DMA,
            pltpu.SemaphoreType.DMA,
        ],
    )
    def comb(ys_hbm, dest_hbm, s1_hbm, out_hbm, idx_v, s1_v,
             r1a, r1b, r2a, r2b, oa, sem_r, sem_w):
        wid = lax.axis_index("s") * 2 + lax.axis_index("c")
        tok0 = wid * TOK_PT
        pltpu.sync_copy(dest_hbm.at[wid], idx_v)
        pltpu.sync_copy(s1_hbm.at[pl.ds(tok0, TOK_PT)], s1_v)
        bufs1 = [r1a, r1b]
        bufs2 = [r2a, r2b]
        g1 = pltpu.async_copy(ys_hbm.at[idx_v.at[0]], r1a, sem_r)
        g2 = pltpu.async_copy(ys_hbm.at[idx_v.at[CHT]], r2a, sem_r)
        wout = None
        for c in range(CHT):
            cur1 = bufs1[c % 2]
            cur2 = bufs2[c % 2]
            g1.wait()
            g2.wait()
            if c + 1 < CHT:
                g1 = pltpu.async_copy(ys_hbm.at[idx_v.at[c + 1]],
                                      bufs1[(c + 1) % 2], sem_r)
                g2 = pltpu.async_copy(ys_hbm.at[idx_v.at[CHT + c + 1]],
                                      bufs2[(c + 1) % 2], sem_r)
            if wout is not None:
                wout.wait()

            def body(r, _):
                s1v = s1_v[c * CBC + r, pl.ds(0, 16)]
                for v in range(D_MODEL // 16):
                    a = cur1[r, pl.ds(v * 16, 16)]
                    b = cur2[r, pl.ds(v * 16, 16)]
                    oa[r, pl.ds(v * 16, 16)] = b + s1v * (a - b)
                return 0

            lax.fori_loop(0, CBC, body, 0)
            wout = pltpu.async_copy(
                oa, out_hbm.at[pl.ds(tok0 + c * CBC, CBC)], sem_w)
        wout.wait()

    return comb


def _ffn_body(be_ref, xs_ref, w1_ref, b1_ref, w2_ref, b2_ref, ys_ref):
    x = xs_ref[...].astype(jnp.bfloat16)
    w1 = w1_ref[0].astype(jnp.bfloat16)
    h = jnp.dot(x, w1, preferred_element_type=jnp.float32)
    h = jnp.maximum(h + b1_ref[0], 0.0).astype(jnp.bfloat16)
    w2 = w2_ref[0].astype(jnp.bfloat16)
    y = jnp.dot(h, w2, preferred_element_type=jnp.float32)
    ys_ref[...] = y + b2_ref[0]


def _ffn(be, xs, w1b, b1, w2b, b2):
    grid_spec = pltpu.PrefetchScalarGridSpec(
        num_scalar_prefetch=1,
        grid=(NB,),
        in_specs=[
            pl.BlockSpec((BM, D_MODEL), lambda i, be_r: (i, 0)),
            pl.BlockSpec((1, D_MODEL, D_FF), lambda i, be_r: (be_r[i], 0, 0)),
            pl.BlockSpec((1, 1, D_FF), lambda i, be_r: (be_r[i], 0, 0)),
            pl.BlockSpec((1, D_FF, D_MODEL), lambda i, be_r: (be_r[i], 0, 0)),
            pl.BlockSpec((1, 1, D_MODEL), lambda i, be_r: (be_r[i], 0, 0)),
        ],
        out_specs=pl.BlockSpec((BM, D_MODEL), lambda i, be_r: (i, 0)),
    )
    return pl.pallas_call(
        _ffn_body,
        grid_spec=grid_spec,
        out_shape=jax.ShapeDtypeStruct((PMAX, D_MODEL), jnp.float32),
        compiler_params=pltpu.CompilerParams(
            dimension_semantics=("arbitrary",),
        ),
    )(be, xs, w1b, b1, w2b, b2)


def kernel(x, Wg, bg, W1, b1, W2, b2):
    wg_pad = jnp.zeros((D_MODEL, E_PAD), jnp.float32).at[:, :NUM_EXPERT].set(Wg)
    bg_pad = jnp.full((1, E_PAD), NEG_BIG, jnp.float32).at[0, :NUM_EXPERT].set(bg)
    e1, e2, sc12 = _gating(x, wg_pad, bg_pad)
    eids8 = jnp.concatenate([e1, e2], axis=0)
    rank8, counts = _rank(eids8)
    dest8, blkexp = _dest(eids8, rank8, counts)
    dcol = dest8[:, 0]
    dlo = dcol[:T]
    dhi = dcol[T:]
    dpair = jnp.concatenate([dlo.reshape(NW, CHD, CB),
                             dhi.reshape(NW, CHD, CB)], axis=1)
    dcomb = jnp.concatenate([dlo.reshape(NW, CHT, CBC),
                             dhi.reshape(NW, CHT, CBC)], axis=1)
    s1b = jnp.broadcast_to(sc12[:, 0:1], (T, 16))
    be = blkexp[0, :NB]

    xs = _make_dispatch()(x, dpair)
    ys = _ffn(be, xs, W1, b1[:, None, :], W2, b2[:, None, :])
    return _make_combine()(ys, dcomb, s1b)
